# Initial kernel scaffold; baseline (speedup 1.0000x reference)
import jax
import jax.numpy as jnp
from jax.experimental import pallas as pl


def kernel(x):
    raise NotImplementedError("write your pallas kernel here")


# speedup vs baseline: 3.0241x; 3.0241x over previous
import functools

import jax
import jax.numpy as jnp
from jax.experimental import pallas as pl

R_BLK = 128
W = 256
N_COLS = 8192


def _cumsum_kernel(x_ref, o_ref):
    idx = jax.lax.iota(jnp.int32, W)
    u = (idx[:, None] <= idx[None, :]).astype(jnp.float32)
    n_chunks = N_COLS // W

    def body(c, carry):
        xc = x_ref[:, pl.ds(c * W, W)]
        y = jax.lax.dot(xc, u, preferred_element_type=jnp.float32) + carry
        o_ref[:, pl.ds(c * W, W)] = y
        return y[:, W - 1:W]

    carry0 = jnp.zeros((R_BLK, 1), jnp.float32)
    jax.lax.fori_loop(0, n_chunks, body, carry0)


@jax.jit
def kernel(x):
    n_rows, n_cols = x.shape
    grid = (n_rows // R_BLK,)
    return pl.pallas_call(
        _cumsum_kernel,
        grid=grid,
        in_specs=[pl.BlockSpec((R_BLK, n_cols), lambda i: (i, 0))],
        out_specs=pl.BlockSpec((R_BLK, n_cols), lambda i: (i, 0)),
        out_shape=jax.ShapeDtypeStruct((n_rows, n_cols), jnp.float32),
    )(x)
